# baseline (speedup 1.0000x reference)
import functools
import numpy as np
import jax
import jax.numpy as jnp
from jax import lax
from jax.experimental import pallas as pl
from jax.experimental.pallas import tpu as pltpu, tpu_sc as plsc

FIELD_DIMS_ = [100000] * 39
_sel = np.hstack((
    np.array(FIELD_DIMS_[0:2]), np.array(FIELD_DIMS_[4:6]), FIELD_DIMS_[12],
    np.array(FIELD_DIMS_[17:21]), np.array(FIELD_DIMS_[26:])))
_OFFSETS = np.array((0, *np.cumsum(_sel)[:-1]), dtype=np.int32)
_COLS = np.array([0, 1, 4, 5, 12, 17, 18, 19, 20] + list(range(26, 39)),
                 dtype=np.int32)

B = 16384
F = 22
D = 16
NC, NS, L = 2, 16, 16
NW = NC * NS
BPW = B // NW
BLK = 128
NBLK = BPW // BLK
NQ = B // BLK
NLINES = 2200000 * D // 128


def _fm_body(lin_hbm, bias_hbm, table_hbm, out_hbm,
             idx_v, voff_v, out_v, bias_v, bufa_v, bufb_v, s_v, q_v,
             sema, semb, semi):
    wid = lax.axis_index("s") * NC + lax.axis_index("c")
    iota = lax.iota(jnp.int32, L)

    for f in range(F):
        pltpu.async_copy(lin_hbm.at[pl.ds(f * NQ + NBLK * wid, NBLK)],
                         idx_v.at[pl.ds(f * NBLK, NBLK)], semi)
    pltpu.sync_copy(bias_hbm, bias_v)
    for f in range(F):
        pltpu.make_async_copy(lin_hbm.at[pl.ds(0, NBLK)],
                              idx_v.at[pl.ds(0, NBLK)], semi).wait()

    def split_body(r, _):
        for c in range(BLK // L):
            v = idx_v[r, pl.ds(c * L, L)]
            voff_v[r, pl.ds(c * L, L)] = (v & 7) << 4
            idx_v[r, pl.ds(c * L, L)] = lax.shift_right_logical(v, 3)
        return 0
    lax.fori_loop(0, F * NBLK, split_body, 0)

    bias_vec = bias_v[pl.ds(0, L)]
    zero16 = jnp.zeros((L,), jnp.float32)

    def start(f, b, buf, sem):
        return pltpu.async_copy(table_hbm.at[idx_v.at[f * NBLK + b]],
                                buf, sem)

    def wait(buf, sem):
        pltpu.make_async_copy(table_hbm.at[idx_v.at[0]], buf, sem).wait()

    def accum(f, b, buf):
        def group_body(g, _):
            voff = voff_v[f * NBLK + b, pl.ds(g * L, L)]
            rows = g * L + iota
            for d in range(D):
                col = plsc.load_gather(buf, [rows, voff + d])
                sl = pl.ds(d * BLK + g * L, L)
                s_v[sl] = s_v[sl] + col
                q_v[sl] = q_v[sl] + col * col
            return 0
        lax.fori_loop(0, BLK // L, group_body, 0)

    def block_body(b, _):
        def zero_body(i, _):
            s_v[pl.ds(i * L, L)] = zero16
            q_v[pl.ds(i * L, L)] = zero16
            return 0
        lax.fori_loop(0, D * BLK // L, zero_body, 0)

        start(0, b, bufa_v, sema)
        start(1, b, bufb_v, semb)

        def pair_body(i, _):
            f0 = 2 * i
            wait(bufa_v, sema)
            accum(f0, b, bufa_v)
            start(f0 + 2, b, bufa_v, sema)
            wait(bufb_v, semb)
            accum(f0 + 1, b, bufb_v)
            start(f0 + 3, b, bufb_v, semb)
            return 0
        lax.fori_loop(0, (F - 2) // 2, pair_body, 0)
        wait(bufa_v, sema)
        accum(F - 2, b, bufa_v)
        wait(bufb_v, semb)
        accum(F - 1, b, bufb_v)

        def out_body(g, _):
            z = zero16
            for d in range(D):
                sl = pl.ds(d * BLK + g * L, L)
                s = s_v[sl]
                q = q_v[sl]
                z = z + (s + 0.5 * (s * s - q))
            y = 1.0 / (1.0 + jnp.exp(-(z + bias_vec)))
            out_v[pl.ds(b * BLK + g * L, L)] = y
            return 0
        lax.fori_loop(0, BLK // L, out_body, 0)
        return 0

    lax.fori_loop(0, NBLK, block_body, 0)
    pltpu.sync_copy(out_v, out_hbm.at[pl.ds(wid * BPW, BPW)])


@jax.jit
def _fm_call(lin2d, bias128, table2d):
    mesh = plsc.VectorSubcoreMesh(core_axis_name="c", subcore_axis_name="s",
                                  num_cores=NC, num_subcores=NS)
    fn = pl.kernel(
        _fm_body,
        out_type=jax.ShapeDtypeStruct((B,), jnp.float32),
        mesh=mesh,
        compiler_params=pltpu.CompilerParams(needs_layout_passes=False,
                                             use_tc_tiling_on_sc=True),
        scratch_types=[
            pltpu.VMEM((F * NBLK, BLK), jnp.int32),
            pltpu.VMEM((F * NBLK, BLK), jnp.int32),
            pltpu.VMEM((BPW,), jnp.float32),
            pltpu.VMEM((128,), jnp.float32),
            pltpu.VMEM((BLK, 128), jnp.float32),
            pltpu.VMEM((BLK, 128), jnp.float32),
            pltpu.VMEM((D * BLK,), jnp.float32),
            pltpu.VMEM((D * BLK,), jnp.float32),
            pltpu.SemaphoreType.DMA,
            pltpu.SemaphoreType.DMA,
            pltpu.SemaphoreType.DMA,
        ],
    )
    return fn(lin2d, bias128, table2d)


def kernel(x, additional, column, emb_table, bias):
    del additional, column
    lin = (x[:, _COLS].astype(jnp.int32)
           + jnp.asarray(_OFFSETS, dtype=jnp.int32)[None, :])
    lin2d = lin.T.reshape(F * NQ, BLK)
    bias128 = jnp.broadcast_to(bias.astype(jnp.float32), (128,))
    table2d = emb_table.reshape(NLINES, 128)
    return _fm_call(lin2d, bias128, table2d)


# speedup vs baseline: 3.6565x; 3.6565x over previous
import numpy as np
import jax
import jax.numpy as jnp
from jax import lax
from jax.experimental import pallas as pl
from jax.experimental.pallas import tpu as pltpu, tpu_sc as plsc

B = 16384
F = 22
D = 16
W = 100000
NC, NS, L = 2, 16, 16
HALF = B // NC
SLICE = HALF // NS
NROWS = 2200000
_STARTS = [(f * W) // 128 * 128 for f in range(F)]
_SHIFT = [f * W - _STARTS[f] for f in range(F)]
SLAB = ((W + 127) // 128 + 1) * 128
_LEN = [SLAB] * (F - 1) + [99968]
TAIL = 64
TAIL_START = NROWS - TAIL
_COLS = [0, 1, 4, 5, 12, 17, 18, 19, 20] + list(range(26, 39))


def _fm_body(xt_hbm, bias_hbm, tab_hbm, tail_hbm, out_hbm,
             slab_v, idx_v, s_v, q_v, out_v, bias_v, tail_v, shared):
    cid = lax.axis_index("c")
    sid = lax.axis_index("s")
    base = cid * HALF

    pltpu.sync_copy(bias_hbm, bias_v)
    pltpu.sync_copy(tail_hbm, tail_v)
    bias_vec = bias_v[pl.ds(0, L)]
    zero_i = jnp.zeros((L,), jnp.int32)

    QT = 4
    QL = HALF // QT

    for f in range(F):
        pltpu.sync_copy(tab_hbm.at[pl.ds(sid, 1), pl.ds(_STARTS[f], _LEN[f])],
                        slab_v.at[:, pl.ds(0, _LEN[f])])
        shift = _SHIFT[f]
        lim = _LEN[f]
        for qt in range(QT):
            pltpu.sync_copy(
                xt_hbm.at[pl.ds(_COLS[f], 1),
                          pl.ds(base + qt * QL, QL)], idx_v)
            qbase = qt * QL

            if f == 0:
                def init_body(g, _):
                    idx = idx_v[0, pl.ds(g * L, L)] + shift
                    val = plsc.load_gather(slab_v, [zero_i, idx])
                    sl = pl.ds(qbase + g * L, L)
                    s_v[0, sl] = val
                    q_v[0, sl] = val * val
                    return 0
                lax.fori_loop(0, QL // L, init_body, 0)
            elif f < F - 1:
                def acc_body(g, _):
                    idx = idx_v[0, pl.ds(g * L, L)] + shift
                    val = plsc.load_gather(slab_v, [zero_i, idx])
                    sl = pl.ds(qbase + g * L, L)
                    s_v[0, sl] = s_v[0, sl] + val
                    q_v[0, sl] = q_v[0, sl] + val * val
                    return 0
                lax.fori_loop(0, QL // L, acc_body, 0)
            else:
                def tail_body(g, _):
                    idx = idx_v[0, pl.ds(g * L, L)] + shift
                    in_slab = idx < lim
                    val_a = plsc.load_gather(
                        slab_v, [zero_i, jnp.minimum(idx, lim - 1)])
                    trow = jnp.clip(idx - lim, 0, TAIL - 1)
                    val_b = plsc.load_gather(tail_v,
                                             [zero_i, sid * TAIL + trow])
                    val = jnp.where(in_slab, val_a, val_b)
                    sl = pl.ds(qbase + g * L, L)
                    s_v[0, sl] = s_v[0, sl] + val
                    q_v[0, sl] = q_v[0, sl] + val * val
                    return 0
                lax.fori_loop(0, QL // L, tail_body, 0)

    def t_body(g, _):
        sl = pl.ds(g * L, L)
        s = s_v[0, sl]
        s_v[0, sl] = s + 0.5 * (s * s - q_v[0, sl])
        return 0
    lax.fori_loop(0, HALF // L, t_body, 0)
    pltpu.sync_copy(s_v, shared.at[pl.ds(sid, 1)])
    plsc.subcore_barrier()

    for d in range(D):
        pltpu.sync_copy(shared.at[pl.ds(d, 1), pl.ds(sid * SLICE, SLICE)],
                        slab_v.at[:, pl.ds(d * SLICE, SLICE)])

    def out_body(g, _):
        z = slab_v[0, pl.ds(g * L, L)]
        for d in range(1, D):
            z = z + slab_v[0, pl.ds(d * SLICE + g * L, L)]
        y = 1.0 / (1.0 + jnp.exp(-(z + bias_vec)))
        out_v[pl.ds(g * L, L)] = y
        return 0
    lax.fori_loop(0, SLICE // L, out_body, 0)

    pltpu.sync_copy(out_v,
                    out_hbm.at[pl.ds(base + sid * SLICE, SLICE)])


@jax.jit
def _fm_call(xt, bias128, tabt, tail):
    mesh = plsc.VectorSubcoreMesh(core_axis_name="c", subcore_axis_name="s",
                                  num_cores=NC, num_subcores=NS)
    fn = pl.kernel(
        _fm_body,
        out_type=jax.ShapeDtypeStruct((B,), jnp.float32),
        mesh=mesh,
        compiler_params=pltpu.CompilerParams(needs_layout_passes=False,
                                             use_tc_tiling_on_sc=True),
        scratch_types=[
            pltpu.VMEM((1, SLAB), jnp.float32),
            pltpu.VMEM((1, HALF // 4), jnp.int32),
            pltpu.VMEM((1, HALF), jnp.float32),
            pltpu.VMEM((1, HALF), jnp.float32),
            pltpu.VMEM((SLICE,), jnp.float32),
            pltpu.VMEM((128,), jnp.float32),
            pltpu.VMEM((1, TAIL * D), jnp.float32),
            pltpu.VMEM_SHARED((NS, HALF), jnp.float32),
        ],
    )
    return fn(xt, bias128, tabt, tail)


def kernel(x, additional, column, emb_table, bias):
    del additional, column
    xt = x.T
    tabt = emb_table.T
    bias128 = jnp.broadcast_to(bias.astype(jnp.float32), (128,))
    tail = emb_table[TAIL_START:, :].T.reshape(1, TAIL * D)
    return _fm_call(xt, bias128, tabt, tail)
